# baseline (speedup 1.0000x reference)
import jax
import jax.numpy as jnp
from jax import lax
from jax.experimental import pallas as pl
from jax.experimental.pallas import tpu as pltpu
from jax.experimental.pallas import tpu_sc as plsc

B = 64
D = 768
H = 12
HD = 64
NG = 6
NE = 6
G_LEN = 5
E_LEN = 5
POOL = 64
TOPK = 8

NTILES = 32
B_SC = 32
B_TC = B - B_SC
ITEMS_CHUNK = B_SC * TOPK
PER_TILE_CHUNK = ITEMS_CHUNK // NTILES


def _topk_kernel(q_ref, k_ref, idx_ref, idx8_ref):
    q = q_ref[...]
    k = k_ref[...]
    qn = q / jnp.maximum(jnp.sqrt(jnp.sum(q * q, axis=1, keepdims=True)), 1e-12)
    kn = k / jnp.maximum(jnp.sqrt(jnp.sum(k * k, axis=1, keepdims=True)), 1e-12)
    sim = jnp.dot(qn, kn.T)
    col = jax.lax.broadcasted_iota(jnp.int32, (B, POOL), 1)
    for t in range(TOPK):
        m = jnp.max(sim, axis=1, keepdims=True)
        amax = jnp.min(jnp.where(sim == m, col, POOL), axis=1)
        idx_ref[:, t] = amax
        idx8_ref[:, t * 8:(t + 1) * 8] = jnp.broadcast_to(amax[:, None], (B, 8))
        sim = jnp.where(col == amax[:, None], -jnp.inf, sim)


def _tc_gather_kernel(idx_ref, pool_ref, g_ref, ek_ref, ev_ref, gk_ref, gv_ref):
    p = pl.program_id(0)
    for t in range(TOPK):
        i = idx_ref[B_SC + p, t]
        for e in range(E_LEN):
            ek_ref[:, 0, :, t * E_LEN + e, :] = pool_ref[i, :, 0, e, :, :]
            ev_ref[:, 0, :, t * E_LEN + e, :] = pool_ref[i, :, 1, e, :, :]
    for e in range(G_LEN):
        for j in range(2):
            gk_ref[:, j, :, e, :] = g_ref[:, 0, e, :, :]
            gv_ref[:, j, :, e, :] = g_ref[:, 1, e, :, :]


def _prep_kernel(pool_ref, poolT_ref):
    for e in range(E_LEN):
        for kv in range(2):
            poolT_ref[0, kv, :, :, e, :] = pool_ref[0, :, kv, e, :, :]


def _make_sc_gather(c):
    def _sc_gather(pool, idx8_flat, ek, ev, idx_v, buf, sem0, sem1):
        wid = lax.axis_index("s") * 2 + lax.axis_index("c")
        base = wid * PER_TILE_CHUNK
        pltpu.sync_copy(
            idx8_flat.at[pl.ds((c * ITEMS_CHUNK + base) * 8, PER_TILE_CHUNK * 8)],
            idx_v)
        sems = (sem0, sem1)

        def start(i):
            par = i % 2
            pltpu.async_copy(pool.at[idx_v.at[pl.ds(i * 8, 1)]], buf.at[par],
                             sems[par])

        start(0)
        for i in range(PER_TILE_CHUNK):
            par = i % 2
            if i + 1 < PER_TILE_CHUNK:
                start(i + 1)
            pltpu.make_async_copy(pool.at[idx_v.at[pl.ds(i * 8, 1)]], buf.at[par],
                                 sems[par]).wait()
            item = base + i
            b = item // TOPK
            t = item % TOPK
            pltpu.sync_copy(buf.at[par, 0, 0],
                            ek.at[:, b, :, pl.ds(t * E_LEN, E_LEN), :])
            pltpu.sync_copy(buf.at[par, 0, 1],
                            ev.at[:, b, :, pl.ds(t * E_LEN, E_LEN), :])

    return _sc_gather


def kernel(query, g_prompt, e_prompt_pool, e_prompt_keys):
    idx, idx8 = pl.pallas_call(
        _topk_kernel,
        out_shape=[
            jax.ShapeDtypeStruct((B, TOPK), jnp.int32),
            jax.ShapeDtypeStruct((B, TOPK * 8), jnp.int32),
        ],
    )(query, e_prompt_keys)

    g_shape = g_prompt.shape
    pool_shape = e_prompt_pool.shape

    poolT = pl.pallas_call(
        _prep_kernel,
        grid=(POOL,),
        in_specs=[
            pl.BlockSpec((1,) + pool_shape[1:], lambda p: (p, 0, 0, 0, 0, 0)),
        ],
        out_specs=pl.BlockSpec((1, 2, NE, H, E_LEN, HD),
                               lambda p: (p, 0, 0, 0, 0, 0)),
        out_shape=jax.ShapeDtypeStruct((POOL, 2, NE, H, E_LEN, HD), jnp.float32),
    )(e_prompt_pool)

    mesh = plsc.VectorSubcoreMesh(core_axis_name="c", subcore_axis_name="s")
    idx8_flat = idx8.reshape(B * TOPK * 8)
    ek_sc, ev_sc = pl.kernel(
        _make_sc_gather(0),
        mesh=mesh,
        compiler_params=pltpu.CompilerParams(use_tc_tiling_on_sc=False),
        out_type=[
            jax.ShapeDtypeStruct((NE, B_SC, H, TOPK * E_LEN, HD), jnp.float32),
            jax.ShapeDtypeStruct((NE, B_SC, H, TOPK * E_LEN, HD), jnp.float32),
        ],
        scratch_types=[
            pltpu.VMEM((PER_TILE_CHUNK * 8,), jnp.int32),
            pltpu.VMEM((2, 1, 2, NE, H, E_LEN, HD), jnp.float32),
            pltpu.SemaphoreType.DMA,
            pltpu.SemaphoreType.DMA,
        ],
    )(poolT, idx8_flat)

    ek_tc, ev_tc, gk, gv = pl.pallas_call(
        _tc_gather_kernel,
        grid_spec=pltpu.PrefetchScalarGridSpec(
            num_scalar_prefetch=1,
            grid=(B_TC,),
            in_specs=[
                pl.BlockSpec(pool_shape, lambda p, idx: (0, 0, 0, 0, 0, 0)),
                pl.BlockSpec(g_shape, lambda p, idx: (0, 0, 0, 0, 0)),
            ],
            out_specs=[
                pl.BlockSpec((NE, 1, H, TOPK * E_LEN, HD),
                             lambda p, idx: (0, p, 0, 0, 0)),
                pl.BlockSpec((NE, 1, H, TOPK * E_LEN, HD),
                             lambda p, idx: (0, p, 0, 0, 0)),
                pl.BlockSpec((NG, 2, H, G_LEN, HD), lambda p, idx: (0, p, 0, 0, 0)),
                pl.BlockSpec((NG, 2, H, G_LEN, HD), lambda p, idx: (0, p, 0, 0, 0)),
            ],
        ),
        out_shape=[
            jax.ShapeDtypeStruct((NE, B_TC, H, TOPK * E_LEN, HD), jnp.float32),
            jax.ShapeDtypeStruct((NE, B_TC, H, TOPK * E_LEN, HD), jnp.float32),
            jax.ShapeDtypeStruct((NG, B, H, G_LEN, HD), jnp.float32),
            jax.ShapeDtypeStruct((NG, B, H, G_LEN, HD), jnp.float32),
        ],
    )(idx, e_prompt_pool, g_prompt)

    ek = jnp.concatenate([ek_sc, ek_tc], axis=1)
    ev = jnp.concatenate([ev_sc, ev_tc], axis=1)
    return gk, gv, ek, ev


# speedup vs baseline: 1.2239x; 1.2239x over previous
import jax
import jax.numpy as jnp
from jax import lax
from jax.experimental import pallas as pl
from jax.experimental.pallas import tpu as pltpu
from jax.experimental.pallas import tpu_sc as plsc

B = 64
D = 768
H = 12
HD = 64
NG = 6
NE = 6
G_LEN = 5
E_LEN = 5
POOL = 64
TOPK = 8

NTILES = 32
PER_TILE = (B * TOPK) // NTILES


def _topk_kernel(q_ref, k_ref, idx_ref, idx8_ref):
    q = q_ref[...]
    k = k_ref[...]
    qn = q / jnp.maximum(jnp.sqrt(jnp.sum(q * q, axis=1, keepdims=True)), 1e-12)
    kn = k / jnp.maximum(jnp.sqrt(jnp.sum(k * k, axis=1, keepdims=True)), 1e-12)
    sim = jnp.dot(qn, kn.T)
    col = jax.lax.broadcasted_iota(jnp.int32, (B, POOL), 1)
    for t in range(TOPK):
        m = jnp.max(sim, axis=1, keepdims=True)
        amax = jnp.min(jnp.where(sim == m, col, POOL), axis=1)
        idx_ref[:, t] = amax
        idx8_ref[:, t * 8:(t + 1) * 8] = jnp.broadcast_to(amax[:, None], (B, 8))
        sim = jnp.where(col == amax[:, None], -jnp.inf, sim)


def _prep_kernel(pool_ref, g_ref, poolT_ref, gk_ref, gv_ref):
    for e in range(E_LEN):
        for kv in range(2):
            poolT_ref[0, kv, :, :, e, :] = pool_ref[0, :, kv, e, :, :]
    for e in range(G_LEN):
        gk_ref[:, 0, :, e, :] = g_ref[:, 0, e, :, :]
        gv_ref[:, 0, :, e, :] = g_ref[:, 1, e, :, :]


def _sc_gather(poolT, idx8_flat, ek, ev, idx_v, buf, sem0, sem1):
    wid = lax.axis_index("s") * 2 + lax.axis_index("c")
    base = wid * PER_TILE
    pltpu.sync_copy(idx8_flat.at[pl.ds(base * 8, PER_TILE * 8)], idx_v)
    sems = (sem0, sem1)

    def start(i):
        par = i % 2
        pltpu.async_copy(poolT.at[idx_v.at[pl.ds(i * 8, 1)]], buf.at[par],
                         sems[par])

    start(0)
    for i in range(PER_TILE):
        par = i % 2
        if i + 1 < PER_TILE:
            start(i + 1)
        pltpu.make_async_copy(poolT.at[idx_v.at[pl.ds(i * 8, 1)]], buf.at[par],
                             sems[par]).wait()
        item = base + i
        b = item // TOPK
        t = item % TOPK
        pltpu.sync_copy(buf.at[par, 0, 0],
                        ek.at[:, b, :, pl.ds(t * E_LEN, E_LEN), :])
        pltpu.sync_copy(buf.at[par, 0, 1],
                        ev.at[:, b, :, pl.ds(t * E_LEN, E_LEN), :])


def kernel(query, g_prompt, e_prompt_pool, e_prompt_keys):
    idx, idx8 = pl.pallas_call(
        _topk_kernel,
        out_shape=[
            jax.ShapeDtypeStruct((B, TOPK), jnp.int32),
            jax.ShapeDtypeStruct((B, TOPK * 8), jnp.int32),
        ],
    )(query, e_prompt_keys)

    pool_shape = e_prompt_pool.shape
    g_shape = g_prompt.shape

    poolT, gk, gv = pl.pallas_call(
        _prep_kernel,
        grid=(POOL,),
        in_specs=[
            pl.BlockSpec((1,) + pool_shape[1:], lambda p: (p, 0, 0, 0, 0, 0)),
            pl.BlockSpec(g_shape, lambda p: (0, 0, 0, 0, 0)),
        ],
        out_specs=[
            pl.BlockSpec((1, 2, NE, H, E_LEN, HD), lambda p: (p, 0, 0, 0, 0, 0)),
            pl.BlockSpec((NG, 1, H, G_LEN, HD), lambda p: (0, p, 0, 0, 0)),
            pl.BlockSpec((NG, 1, H, G_LEN, HD), lambda p: (0, p, 0, 0, 0)),
        ],
        out_shape=[
            jax.ShapeDtypeStruct((POOL, 2, NE, H, E_LEN, HD), jnp.float32),
            jax.ShapeDtypeStruct((NG, B, H, G_LEN, HD), jnp.float32),
            jax.ShapeDtypeStruct((NG, B, H, G_LEN, HD), jnp.float32),
        ],
    )(e_prompt_pool, g_prompt)

    mesh = plsc.VectorSubcoreMesh(core_axis_name="c", subcore_axis_name="s")
    ek, ev = pl.kernel(
        _sc_gather,
        mesh=mesh,
        compiler_params=pltpu.CompilerParams(use_tc_tiling_on_sc=False),
        out_type=[
            jax.ShapeDtypeStruct((NE, B, H, TOPK * E_LEN, HD), jnp.float32),
            jax.ShapeDtypeStruct((NE, B, H, TOPK * E_LEN, HD), jnp.float32),
        ],
        scratch_types=[
            pltpu.VMEM((PER_TILE * 8,), jnp.int32),
            pltpu.VMEM((2, 1, 2, NE, H, E_LEN, HD), jnp.float32),
            pltpu.SemaphoreType.DMA,
            pltpu.SemaphoreType.DMA,
        ],
    )(poolT, idx8.reshape(B * TOPK * 8))

    return gk, gv, ek, ev
